# baseline (speedup 1.0000x reference)
import math

import jax
import jax.numpy as jnp
from jax import lax
from jax.experimental import pallas as pl
from jax.experimental.pallas import tpu as pltpu
from jax.experimental.pallas import tpu_sc as plsc

KSEL = 8
NRAY = 8
NF = 512
NPTS = 10000
NPAD = 10240
BF = 8
NSEL = NF * NRAY * KSEL
DPAD = 16


NCH = NPAD // 128


def _dist_topk_body(ro_ref, rd_ref, pts_ref, dist_ref, idx_ref):
    px = pts_ref[0:1, :, :]
    py = pts_ref[1:2, :, :]
    pz = pts_ref[2:3, :, :]
    ox = ro_ref[:, 0:1].reshape(BF, 1, 1)
    oy = ro_ref[:, 1:2].reshape(BF, 1, 1)
    oz = ro_ref[:, 2:3].reshape(BF, 1, 1)
    dx = px - ox
    dy = py - oy
    dz = pz - oz
    dnorm = jnp.sqrt((dx * dx + dy * dy) + dz * dz)
    maxd = jnp.maximum(dnorm, 1e-12)
    ux = dx / maxd
    uy = dy / maxd
    uz = dz / maxd
    flat = (lax.broadcasted_iota(jnp.int32, (BF, NCH, 128), 1) * 128
            + lax.broadcasted_iota(jnp.int32, (BF, NCH, 128), 2))
    colmask = flat < NPTS
    big = jnp.int32(2 ** 30)
    inf = jnp.float32(jnp.inf)
    for r in range(NRAY):
        a = rd_ref[:, 3 * r:3 * r + 1].reshape(BF, 1, 1)
        b = rd_ref[:, 3 * r + 1:3 * r + 2].reshape(BF, 1, 1)
        c = rd_ref[:, 3 * r + 2:3 * r + 3].reshape(BF, 1, 1)
        nrm = jnp.sqrt((a * a + b * b) + c * c)
        nrm = jnp.maximum(nrm, 1e-12)
        an = a / nrm
        bn = b / nrm
        cn = c / nrm
        cos = (an * ux + bn * uy) + cn * uz
        sinphi = jnp.sqrt(jnp.maximum(1.0 - cos * cos, 1e-12))
        proj = sinphi * dnorm
        proj = jnp.where(cos < 0.866, 1e8, proj)
        p = jnp.where(colmask, proj, inf)

        m1 = jnp.min(p, axis=1)
        f1 = jnp.min(jnp.where(p == m1[:, None, :], flat, big), axis=1)
        rest = jnp.where(flat == f1[:, None, :], inf, p)
        m2 = jnp.min(rest, axis=1)
        f2 = jnp.min(jnp.where(rest == m2[:, None, :], flat, big), axis=1)

        c1v, c1f, c2v, c2f = [], [], [], []
        wv = m1
        for k in range(KSEL):
            mv = jnp.min(wv, axis=1, keepdims=True)
            mf = jnp.min(jnp.where(wv == mv, f1, big), axis=1, keepdims=True)
            sel = f1 == mf
            c1v.append(mv)
            c1f.append(mf)
            c2v.append(jnp.min(jnp.where(sel, m2, inf), axis=1, keepdims=True))
            c2f.append(jnp.min(jnp.where(sel, f2, big), axis=1, keepdims=True))
            wv = jnp.where(sel, inf, wv)
        v16 = jnp.concatenate(c1v + c2v, axis=1)
        f16 = jnp.concatenate(c1f + c2f, axis=1)

        dvals, ivals = [], []
        for k in range(KSEL):
            mv = jnp.min(v16, axis=1, keepdims=True)
            mf = jnp.min(jnp.where(v16 == mv, f16, big), axis=1, keepdims=True)
            dvals.append(mv)
            ivals.append(mf)
            v16 = jnp.where(f16 == mf, inf, v16)
        dist_ref[:, r, :] = jnp.concatenate(dvals, axis=1)
        i8 = jnp.concatenate(ivals, axis=1)
        idx_ref[:, r, :] = i8

        bad = jnp.zeros((BF, 1), dtype=jnp.bool_)
        for j in range(KSEL):
            s1 = jnp.any(i8 == c1f[j], axis=1, keepdims=True)
            s2 = jnp.any(i8 == c2f[j], axis=1, keepdims=True)
            bad = bad | (s1 & s2)

        @pl.when(jnp.any(bad))
        def _fallback(p=p, r=r):
            dv, iv = [], []
            for k in range(KSEL):
                ml = jnp.min(p, axis=1)
                mv = jnp.min(ml, axis=1, keepdims=True)
                eq = p == mv.reshape(BF, 1, 1)
                fl = jnp.min(jnp.where(eq, flat, big), axis=1)
                mf = jnp.min(fl, axis=1, keepdims=True)
                dv.append(mv)
                iv.append(mf)
                p = jnp.where(flat == mf.reshape(BF, 1, 1), inf, p)
            dist_ref[:, r, :] = jnp.concatenate(dv, axis=1)
            idx_ref[:, r, :] = jnp.concatenate(iv, axis=1)


def _dist_topk(ray_o, ray_d24, pts_t):
    return pl.pallas_call(
        _dist_topk_body,
        grid=(NF // BF,),
        in_specs=[
            pl.BlockSpec((BF, 3), lambda i: (i, 0)),
            pl.BlockSpec((BF, 24), lambda i: (i, 0)),
            pl.BlockSpec((3, NCH, 128), lambda i: (0, 0, 0)),
        ],
        out_specs=[
            pl.BlockSpec((BF, NRAY, KSEL), lambda i: (i, 0, 0)),
            pl.BlockSpec((BF, NRAY, KSEL), lambda i: (i, 0, 0)),
        ],
        out_shape=[
            jax.ShapeDtypeStruct((NF, NRAY, KSEL), jnp.float32),
            jax.ShapeDtypeStruct((NF, NRAY, KSEL), jnp.int32),
        ],
    )(ray_o, ray_d24, pts_t)


_NC = 2
_NS = 16
_NW = _NC * _NS
_BPW = NSEL // _NW


def _gather_body(pts_hbm, idx_hbm, out_hbm, idx_v, rows_v, sem):
    wid = lax.axis_index("s") * _NC + lax.axis_index("c")
    base = wid * _BPW
    pltpu.sync_copy(idx_hbm.at[pl.ds(base, _BPW)], idx_v)
    pltpu.async_copy(pts_hbm.at[idx_v], rows_v, sem).wait()
    pltpu.sync_copy(rows_v, out_hbm.at[pl.ds(base, _BPW)])


def _gather_sc(pts16, idx_flat):
    return pl.kernel(
        _gather_body,
        out_type=jax.ShapeDtypeStruct((NSEL, DPAD), jnp.float32),
        mesh=plsc.VectorSubcoreMesh(core_axis_name="c", subcore_axis_name="s"),
        scratch_types=[
            pltpu.VMEM((_BPW,), jnp.int32),
            pltpu.VMEM((_BPW, DPAD), jnp.float32),
            pltpu.SemaphoreType.DMA,
        ],
        compiler_params=pltpu.CompilerParams(use_tc_tiling_on_sc=False),
    )(pts16, idx_flat)


def _acos(x):
    a = jnp.abs(x)
    p = jnp.float32(-0.0012624911)
    for coef in (0.0066700901, -0.0170881256, 0.0308918810, -0.0501743046,
                 0.0889789874, -0.2145988016, 1.5707963050):
        p = p * a + jnp.float32(coef)
    r = jnp.sqrt(jnp.maximum(1.0 - a, 0.0)) * p
    return jnp.where(x < 0.0, jnp.float32(math.pi) - r, r)


def _feat_body(gx_ref, gy_ref, gz_ref, ro_ref, dist_ref,
               npd_ref, elev_ref, azim_ref, sky_ref):
    eps = 1e-5
    gx = gx_ref[...] - ro_ref[:, 0:1]
    gy = gy_ref[...] - ro_ref[:, 1:2]
    gz = gz_ref[...] - ro_ref[:, 2:3]
    npd = jnp.sqrt((gx * gx + gy * gy) + gz * gz)
    u1 = gz / (npd + eps)
    elev = _acos(u1)
    sin_elev = jnp.sqrt(jnp.maximum(1.0 - u1 * u1, 0.0))
    az = _acos(gx / (npd * sin_elev + eps))
    az = jnp.where(gy < 0.0, 2.0 * math.pi - az, az)
    npd_ref[...] = npd
    elev_ref[...] = elev
    azim_ref[...] = az
    sky_ref[...] = (dist_ref[...] >= 1e8 - 1).astype(jnp.int32)


def _features(gx, gy, gz, ray_o, dist):
    n = NRAY * KSEL
    return pl.pallas_call(
        _feat_body,
        grid=(1,),
        in_specs=[
            pl.BlockSpec((NF, n), lambda i: (0, 0)),
            pl.BlockSpec((NF, n), lambda i: (0, 0)),
            pl.BlockSpec((NF, n), lambda i: (0, 0)),
            pl.BlockSpec((NF, 3), lambda i: (0, 0)),
            pl.BlockSpec((NF, n), lambda i: (0, 0)),
        ],
        out_specs=[pl.BlockSpec((NF, n), lambda i: (0, 0))] * 4,
        out_shape=[
            jax.ShapeDtypeStruct((NF, n), jnp.float32),
            jax.ShapeDtypeStruct((NF, n), jnp.float32),
            jax.ShapeDtypeStruct((NF, n), jnp.float32),
            jax.ShapeDtypeStruct((NF, n), jnp.int32),
        ],
    )(gx, gy, gz, ray_o, dist)


def kernel(ray_o, ray_d, pts):
    ray_d24 = ray_d.reshape(NF, NRAY * 3)
    pts_t = jnp.pad(pts.T, ((0, 0), (0, NPAD - NPTS))).reshape(3, NCH, 128)
    dist, idx = _dist_topk(ray_o, ray_d24, pts_t)

    pts16 = jnp.pad(pts, ((0, 0), (0, DPAD - 3)))
    rows = _gather_sc(pts16, idx.reshape(NSEL))

    g = rows[:, :3].reshape(NF, NRAY * KSEL, 3)
    npd, elev, azim, sky = _features(
        g[:, :, 0], g[:, :, 1], g[:, :, 2], ray_o,
        dist.reshape(NF, NRAY * KSEL))

    shp = (NF, NRAY, KSEL, 1)
    return (dist, idx, sky.astype(bool).reshape(NF, NRAY, KSEL),
            npd.reshape(shp), elev.reshape(shp), azim.reshape(shp))


# speedup vs baseline: 3.6380x; 3.6380x over previous
import math

import jax
import jax.numpy as jnp
from jax import lax
from jax.experimental import pallas as pl
from jax.experimental.pallas import tpu as pltpu
from jax.experimental.pallas import tpu_sc as plsc

KSEL = 8
NRAY = 8
NF = 512
NPTS = 10000
NPAD = 10240
BF = 8
NSEL = NF * NRAY * KSEL
DPAD = 16


NCH = NPAD // 128


def _dist_topk_body(ro_ref, rd_ref, pts_ref, dist_ref, idx_ref, bad_ref):
    px = pts_ref[0:1, :, :]
    py = pts_ref[1:2, :, :]
    pz = pts_ref[2:3, :, :]
    ox = ro_ref[:, 0:1].reshape(BF, 1, 1)
    oy = ro_ref[:, 1:2].reshape(BF, 1, 1)
    oz = ro_ref[:, 2:3].reshape(BF, 1, 1)
    dx = px - ox
    dy = py - oy
    dz = pz - oz
    dnorm = jnp.sqrt((dx * dx + dy * dy) + dz * dz)
    maxd = jnp.maximum(dnorm, 1e-12)
    ux = dx / maxd
    uy = dy / maxd
    uz = dz / maxd
    flat = (lax.broadcasted_iota(jnp.int32, (BF, NCH, 128), 1) * 128
            + lax.broadcasted_iota(jnp.int32, (BF, NCH, 128), 2))
    colmask = flat < NPTS
    big = jnp.int32(2 ** 30)
    inf = jnp.float32(jnp.inf)
    for r in range(NRAY):
        a = rd_ref[:, 3 * r:3 * r + 1].reshape(BF, 1, 1)
        b = rd_ref[:, 3 * r + 1:3 * r + 2].reshape(BF, 1, 1)
        c = rd_ref[:, 3 * r + 2:3 * r + 3].reshape(BF, 1, 1)
        nrm = jnp.sqrt((a * a + b * b) + c * c)
        nrm = jnp.maximum(nrm, 1e-12)
        an = a / nrm
        bn = b / nrm
        cn = c / nrm
        cos = (an * ux + bn * uy) + cn * uz
        sinphi = jnp.sqrt(jnp.maximum(1.0 - cos * cos, 1e-12))
        proj = sinphi * dnorm
        proj = jnp.where(cos < 0.866, 1e8, proj)
        p = jnp.where(colmask, proj, inf)

        m1 = jnp.min(p, axis=1)
        f1 = jnp.min(jnp.where(p == m1[:, None, :], flat, big), axis=1)
        r1 = jnp.where(flat == f1[:, None, :], inf, p)
        m2 = jnp.min(r1, axis=1)
        f2 = jnp.min(jnp.where(r1 == m2[:, None, :], flat, big), axis=1)
        r2 = jnp.where(flat == f2[:, None, :], inf, r1)
        m3 = jnp.min(r2, axis=1)
        f3 = jnp.min(jnp.where(r2 == m3[:, None, :], flat, big), axis=1)

        c1v, c1f, c2v, c2f, c3v, c3f = [], [], [], [], [], []
        wv = m1
        for k in range(KSEL):
            mv = jnp.min(wv, axis=1, keepdims=True)
            mf = jnp.min(jnp.where(wv == mv, f1, big), axis=1, keepdims=True)
            sel = f1 == mf
            c1v.append(mv)
            c1f.append(mf)
            c2v.append(jnp.min(jnp.where(sel, m2, inf), axis=1, keepdims=True))
            c2f.append(jnp.min(jnp.where(sel, f2, big), axis=1, keepdims=True))
            c3v.append(jnp.min(jnp.where(sel, m3, inf), axis=1, keepdims=True))
            c3f.append(jnp.min(jnp.where(sel, f3, big), axis=1, keepdims=True))
            wv = jnp.where(sel, inf, wv)
        v24 = jnp.concatenate(c1v + c2v + c3v, axis=1)
        f24 = jnp.concatenate(c1f + c2f + c3f, axis=1)

        dvals, ivals = [], []
        for k in range(KSEL):
            mv = jnp.min(v24, axis=1, keepdims=True)
            mf = jnp.min(jnp.where(v24 == mv, f24, big), axis=1, keepdims=True)
            dvals.append(mv)
            ivals.append(mf)
            v24 = jnp.where(f24 == mf, inf, v24)
        dist_ref[:, r, :] = jnp.concatenate(dvals, axis=1)
        i8 = jnp.concatenate(ivals, axis=1)
        idx_ref[:, r, :] = i8

        bad = jnp.zeros((BF, 1), dtype=jnp.int32)
        for j in range(KSEL):
            s1 = jnp.any(i8 == c1f[j], axis=1, keepdims=True)
            s2 = jnp.any(i8 == c2f[j], axis=1, keepdims=True)
            s3 = jnp.any(i8 == c3f[j], axis=1, keepdims=True)
            bad = bad | (s1 & s2 & s3).astype(jnp.int32)
        bad_ref[:, r:r + 1] = bad


def _dist_topk(ray_o, ray_d24, pts_t):
    return pl.pallas_call(
        _dist_topk_body,
        grid=(NF // BF,),
        in_specs=[
            pl.BlockSpec((BF, 3), lambda i: (i, 0)),
            pl.BlockSpec((BF, 24), lambda i: (i, 0)),
            pl.BlockSpec((3, NCH, 128), lambda i: (0, 0, 0)),
        ],
        out_specs=[
            pl.BlockSpec((BF, NRAY, KSEL), lambda i: (i, 0, 0)),
            pl.BlockSpec((BF, NRAY, KSEL), lambda i: (i, 0, 0)),
            pl.BlockSpec((BF, NRAY), lambda i: (i, 0)),
        ],
        out_shape=[
            jax.ShapeDtypeStruct((NF, NRAY, KSEL), jnp.float32),
            jax.ShapeDtypeStruct((NF, NRAY, KSEL), jnp.int32),
            jax.ShapeDtypeStruct((NF, NRAY), jnp.int32),
        ],
    )(ray_o, ray_d24, pts_t)


def _repair_body(flags_ref, ro_ref, rd_ref, pts_ref, din_ref, iin_ref,
                 dout_ref, iout_ref):
    dout_ref[...] = din_ref[...]
    iout_ref[...] = iin_ref[...]
    flag = flags_ref[pl.program_id(0), pl.program_id(1)]

    @pl.when(flag > 0)
    def _():
        px = pts_ref[0:1, :, :]
        py = pts_ref[1:2, :, :]
        pz = pts_ref[2:3, :, :]
        ox = ro_ref[:, 0:1].reshape(BF, 1, 1)
        oy = ro_ref[:, 1:2].reshape(BF, 1, 1)
        oz = ro_ref[:, 2:3].reshape(BF, 1, 1)
        dx = px - ox
        dy = py - oy
        dz = pz - oz
        dnorm = jnp.sqrt((dx * dx + dy * dy) + dz * dz)
        maxd = jnp.maximum(dnorm, 1e-12)
        ux = dx / maxd
        uy = dy / maxd
        uz = dz / maxd
        flat = (lax.broadcasted_iota(jnp.int32, (BF, NCH, 128), 1) * 128
                + lax.broadcasted_iota(jnp.int32, (BF, NCH, 128), 2))
        colmask = flat < NPTS
        big = jnp.int32(2 ** 30)
        inf = jnp.float32(jnp.inf)
        a = rd_ref[0, :, 0:1].reshape(BF, 1, 1)
        b = rd_ref[0, :, 1:2].reshape(BF, 1, 1)
        c = rd_ref[0, :, 2:3].reshape(BF, 1, 1)
        nrm = jnp.sqrt((a * a + b * b) + c * c)
        nrm = jnp.maximum(nrm, 1e-12)
        an = a / nrm
        bn = b / nrm
        cn = c / nrm
        cos = (an * ux + bn * uy) + cn * uz
        sinphi = jnp.sqrt(jnp.maximum(1.0 - cos * cos, 1e-12))
        proj = sinphi * dnorm
        proj = jnp.where(cos < 0.866, 1e8, proj)
        p = jnp.where(colmask, proj, inf)
        dv, iv = [], []
        for k in range(KSEL):
            ml = jnp.min(p, axis=1)
            mv = jnp.min(ml, axis=1, keepdims=True)
            eq = p == mv.reshape(BF, 1, 1)
            fl = jnp.min(jnp.where(eq, flat, big), axis=1)
            mf = jnp.min(fl, axis=1, keepdims=True)
            dv.append(mv)
            iv.append(mf)
            p = jnp.where(flat == mf.reshape(BF, 1, 1), inf, p)
        dout_ref[0, :, :] = jnp.concatenate(dv, axis=1)
        iout_ref[0, :, :] = jnp.concatenate(iv, axis=1)


def _repair(gflags, ray_o, rd_t, pts_t, dist_t, idx_t):
    grid_spec = pltpu.PrefetchScalarGridSpec(
        num_scalar_prefetch=1,
        grid=(NF // BF, NRAY),
        in_specs=[
            pl.BlockSpec((BF, 3), lambda i, r, f: (i, 0)),
            pl.BlockSpec((1, BF, 3), lambda i, r, f: (r, i, 0)),
            pl.BlockSpec((3, NCH, 128), lambda i, r, f: (0, 0, 0)),
            pl.BlockSpec((1, BF, KSEL), lambda i, r, f: (r, i, 0)),
            pl.BlockSpec((1, BF, KSEL), lambda i, r, f: (r, i, 0)),
        ],
        out_specs=[
            pl.BlockSpec((1, BF, KSEL), lambda i, r, f: (r, i, 0)),
            pl.BlockSpec((1, BF, KSEL), lambda i, r, f: (r, i, 0)),
        ],
    )
    return pl.pallas_call(
        _repair_body,
        grid_spec=grid_spec,
        out_shape=[
            jax.ShapeDtypeStruct((NRAY, NF, KSEL), jnp.float32),
            jax.ShapeDtypeStruct((NRAY, NF, KSEL), jnp.int32),
        ],
    )(gflags, ray_o, rd_t, pts_t, dist_t, idx_t)


_NC = 2
_NS = 16
_NW = _NC * _NS
_BPW = NSEL // _NW


def _gather_body(pts_hbm, idx_hbm, out_hbm, idx_v, rows_v, sem):
    wid = lax.axis_index("s") * _NC + lax.axis_index("c")
    base = wid * _BPW
    pltpu.sync_copy(idx_hbm.at[pl.ds(base, _BPW)], idx_v)
    pltpu.async_copy(pts_hbm.at[idx_v], rows_v, sem).wait()
    pltpu.sync_copy(rows_v, out_hbm.at[pl.ds(base, _BPW)])


def _gather_sc(pts16, idx_flat):
    return pl.kernel(
        _gather_body,
        out_type=jax.ShapeDtypeStruct((NSEL, DPAD), jnp.float32),
        mesh=plsc.VectorSubcoreMesh(core_axis_name="c", subcore_axis_name="s"),
        scratch_types=[
            pltpu.VMEM((_BPW,), jnp.int32),
            pltpu.VMEM((_BPW, DPAD), jnp.float32),
            pltpu.SemaphoreType.DMA,
        ],
        compiler_params=pltpu.CompilerParams(use_tc_tiling_on_sc=False),
    )(pts16, idx_flat)


def _acos(x):
    a = jnp.abs(x)
    p = jnp.float32(-0.0012624911)
    for coef in (0.0066700901, -0.0170881256, 0.0308918810, -0.0501743046,
                 0.0889789874, -0.2145988016, 1.5707963050):
        p = p * a + jnp.float32(coef)
    r = jnp.sqrt(jnp.maximum(1.0 - a, 0.0)) * p
    return jnp.where(x < 0.0, jnp.float32(math.pi) - r, r)


def _feat_body(gx_ref, gy_ref, gz_ref, ro_ref, dist_ref,
               npd_ref, elev_ref, azim_ref, sky_ref):
    eps = 1e-5
    gx = gx_ref[...] - ro_ref[:, 0:1]
    gy = gy_ref[...] - ro_ref[:, 1:2]
    gz = gz_ref[...] - ro_ref[:, 2:3]
    npd = jnp.sqrt((gx * gx + gy * gy) + gz * gz)
    u1 = gz / (npd + eps)
    elev = _acos(u1)
    sin_elev = jnp.sqrt(jnp.maximum(1.0 - u1 * u1, 0.0))
    az = _acos(gx / (npd * sin_elev + eps))
    az = jnp.where(gy < 0.0, 2.0 * math.pi - az, az)
    npd_ref[...] = npd
    elev_ref[...] = elev
    azim_ref[...] = az
    sky_ref[...] = (dist_ref[...] >= 1e8 - 1).astype(jnp.int32)


def _features(gx, gy, gz, ray_o, dist):
    n = NRAY * KSEL
    return pl.pallas_call(
        _feat_body,
        grid=(1,),
        in_specs=[
            pl.BlockSpec((NF, n), lambda i: (0, 0)),
            pl.BlockSpec((NF, n), lambda i: (0, 0)),
            pl.BlockSpec((NF, n), lambda i: (0, 0)),
            pl.BlockSpec((NF, 3), lambda i: (0, 0)),
            pl.BlockSpec((NF, n), lambda i: (0, 0)),
        ],
        out_specs=[pl.BlockSpec((NF, n), lambda i: (0, 0))] * 4,
        out_shape=[
            jax.ShapeDtypeStruct((NF, n), jnp.float32),
            jax.ShapeDtypeStruct((NF, n), jnp.float32),
            jax.ShapeDtypeStruct((NF, n), jnp.float32),
            jax.ShapeDtypeStruct((NF, n), jnp.int32),
        ],
    )(gx, gy, gz, ray_o, dist)


def kernel(ray_o, ray_d, pts):
    ray_d24 = ray_d.reshape(NF, NRAY * 3)
    pts_t = jnp.pad(pts.T, ((0, 0), (0, NPAD - NPTS))).reshape(3, NCH, 128)
    dist, idx, bad = _dist_topk(ray_o, ray_d24, pts_t)
    gflags = jnp.max(bad.reshape(NF // BF, BF, NRAY), axis=1)
    dist_t, idx_t = _repair(gflags, ray_o, ray_d.transpose(1, 0, 2), pts_t,
                            dist.transpose(1, 0, 2), idx.transpose(1, 0, 2))
    dist = dist_t.transpose(1, 0, 2)
    idx = idx_t.transpose(1, 0, 2)

    pts16 = jnp.pad(pts, ((0, 0), (0, DPAD - 3)))
    rows = _gather_sc(pts16, idx.reshape(NSEL))

    g = rows[:, :3].reshape(NF, NRAY * KSEL, 3)
    npd, elev, azim, sky = _features(
        g[:, :, 0], g[:, :, 1], g[:, :, 2], ray_o,
        dist.reshape(NF, NRAY * KSEL))

    shp = (NF, NRAY, KSEL, 1)
    return (dist, idx, sky.astype(bool).reshape(NF, NRAY, KSEL),
            npd.reshape(shp), elev.reshape(shp), azim.reshape(shp))


# speedup vs baseline: 4.2770x; 1.1756x over previous
import math

import jax
import jax.numpy as jnp
from jax import lax
from jax.experimental import pallas as pl
from jax.experimental.pallas import tpu as pltpu
from jax.experimental.pallas import tpu_sc as plsc

KSEL = 8
NRAY = 8
NF = 512
NPTS = 10000
NPAD = 10240
BF = 8
NSEL = NF * NRAY * KSEL
DPAD = 16


NCH = NPAD // 128


def _dist_topk_body(ro_ref, rd_ref, pts_ref, dist_ref, idx_ref, bad_ref):
    px = pts_ref[0:1, :, :]
    py = pts_ref[1:2, :, :]
    pz = pts_ref[2:3, :, :]
    ox = ro_ref[:, 0:1].reshape(BF, 1, 1)
    oy = ro_ref[:, 1:2].reshape(BF, 1, 1)
    oz = ro_ref[:, 2:3].reshape(BF, 1, 1)
    dx = px - ox
    dy = py - oy
    dz = pz - oz
    dnorm = jnp.sqrt((dx * dx + dy * dy) + dz * dz)
    maxd = jnp.maximum(dnorm, 1e-12)
    ux = dx / maxd
    uy = dy / maxd
    uz = dz / maxd
    flat = (lax.broadcasted_iota(jnp.int32, (BF, NCH, 128), 1) * 128
            + lax.broadcasted_iota(jnp.int32, (BF, NCH, 128), 2))
    colmask = flat < NPTS
    big = jnp.int32(2 ** 30)
    inf = jnp.float32(jnp.inf)
    for r in range(NRAY):
        a = rd_ref[:, 3 * r:3 * r + 1].reshape(BF, 1, 1)
        b = rd_ref[:, 3 * r + 1:3 * r + 2].reshape(BF, 1, 1)
        c = rd_ref[:, 3 * r + 2:3 * r + 3].reshape(BF, 1, 1)
        nrm = jnp.sqrt((a * a + b * b) + c * c)
        nrm = jnp.maximum(nrm, 1e-12)
        an = a / nrm
        bn = b / nrm
        cn = c / nrm
        cos = (an * ux + bn * uy) + cn * uz
        sinphi = jnp.sqrt(jnp.maximum(1.0 - cos * cos, 1e-12))
        proj = sinphi * dnorm
        proj = jnp.where(cos < 0.866, 1e8, proj)
        p = jnp.where(colmask, proj, inf)

        m1 = jnp.min(p, axis=1)
        f1 = jnp.min(jnp.where(p == m1[:, None, :], flat, big), axis=1)
        r1 = jnp.where(flat == f1[:, None, :], inf, p)
        m2 = jnp.min(r1, axis=1)
        f2 = jnp.min(jnp.where(r1 == m2[:, None, :], flat, big), axis=1)
        r2 = jnp.where(flat == f2[:, None, :], inf, r1)
        m3 = jnp.min(r2, axis=1)
        f3 = jnp.min(jnp.where(r2 == m3[:, None, :], flat, big), axis=1)

        c1v, c1f, c2v, c2f, c3v, c3f = [], [], [], [], [], []
        wv = m1
        for k in range(KSEL):
            mv = jnp.min(wv, axis=1, keepdims=True)
            mf = jnp.min(jnp.where(wv == mv, f1, big), axis=1, keepdims=True)
            sel = f1 == mf
            c1v.append(mv)
            c1f.append(mf)
            c2v.append(jnp.min(jnp.where(sel, m2, inf), axis=1, keepdims=True))
            c2f.append(jnp.min(jnp.where(sel, f2, big), axis=1, keepdims=True))
            c3v.append(jnp.min(jnp.where(sel, m3, inf), axis=1, keepdims=True))
            c3f.append(jnp.min(jnp.where(sel, f3, big), axis=1, keepdims=True))
            wv = jnp.where(sel, inf, wv)
        v24 = jnp.concatenate(c1v + c2v + c3v, axis=1)
        f24 = jnp.concatenate(c1f + c2f + c3f, axis=1)

        dvals, ivals = [], []
        for k in range(KSEL):
            mv = jnp.min(v24, axis=1, keepdims=True)
            mf = jnp.min(jnp.where(v24 == mv, f24, big), axis=1, keepdims=True)
            dvals.append(mv)
            ivals.append(mf)
            v24 = jnp.where(f24 == mf, inf, v24)
        dist_ref[:, r, :] = jnp.concatenate(dvals, axis=1)
        i8 = jnp.concatenate(ivals, axis=1)
        idx_ref[:, r, :] = i8

        bad = jnp.zeros((BF, 1), dtype=jnp.int32)
        for j in range(KSEL):
            s1 = jnp.any(i8 == c1f[j], axis=1, keepdims=True)
            s2 = jnp.any(i8 == c2f[j], axis=1, keepdims=True)
            s3 = jnp.any(i8 == c3f[j], axis=1, keepdims=True)
            bad = bad | (s1 & s2 & s3).astype(jnp.int32)
        bad_ref[:, r:r + 1] = bad


def _dist_topk(ray_o, ray_d24, pts_t):
    return pl.pallas_call(
        _dist_topk_body,
        grid=(NF // BF,),
        in_specs=[
            pl.BlockSpec((BF, 3), lambda i: (i, 0)),
            pl.BlockSpec((BF, 24), lambda i: (i, 0)),
            pl.BlockSpec((3, NCH, 128), lambda i: (0, 0, 0)),
        ],
        out_specs=[
            pl.BlockSpec((BF, NRAY, KSEL), lambda i: (i, 0, 0)),
            pl.BlockSpec((BF, NRAY, KSEL), lambda i: (i, 0, 0)),
            pl.BlockSpec((BF, NRAY), lambda i: (i, 0)),
        ],
        out_shape=[
            jax.ShapeDtypeStruct((NF, NRAY, KSEL), jnp.float32),
            jax.ShapeDtypeStruct((NF, NRAY, KSEL), jnp.int32),
            jax.ShapeDtypeStruct((NF, NRAY), jnp.int32),
        ],
    )(ray_o, ray_d24, pts_t)


def _repair_body(ids_ref, flags_ref, ro_ref, rd_ref, pts_ref, din_ref,
                 iin_ref, dout_ref, iout_ref):
    dout_ref[...] = din_ref[...]
    iout_ref[...] = iin_ref[...]
    flag = flags_ref[pl.program_id(0)]

    @pl.when(flag > 0)
    def _():
        px = pts_ref[0:1, :, :]
        py = pts_ref[1:2, :, :]
        pz = pts_ref[2:3, :, :]
        ox = ro_ref[:, 0:1].reshape(BF, 1, 1)
        oy = ro_ref[:, 1:2].reshape(BF, 1, 1)
        oz = ro_ref[:, 2:3].reshape(BF, 1, 1)
        dx = px - ox
        dy = py - oy
        dz = pz - oz
        dnorm = jnp.sqrt((dx * dx + dy * dy) + dz * dz)
        maxd = jnp.maximum(dnorm, 1e-12)
        ux = dx / maxd
        uy = dy / maxd
        uz = dz / maxd
        flat = (lax.broadcasted_iota(jnp.int32, (BF, NCH, 128), 1) * 128
                + lax.broadcasted_iota(jnp.int32, (BF, NCH, 128), 2))
        colmask = flat < NPTS
        big = jnp.int32(2 ** 30)
        inf = jnp.float32(jnp.inf)
        for r in range(NRAY):
            a = rd_ref[:, 3 * r:3 * r + 1].reshape(BF, 1, 1)
            b = rd_ref[:, 3 * r + 1:3 * r + 2].reshape(BF, 1, 1)
            c = rd_ref[:, 3 * r + 2:3 * r + 3].reshape(BF, 1, 1)
            nrm = jnp.sqrt((a * a + b * b) + c * c)
            nrm = jnp.maximum(nrm, 1e-12)
            an = a / nrm
            bn = b / nrm
            cn = c / nrm
            cos = (an * ux + bn * uy) + cn * uz
            sinphi = jnp.sqrt(jnp.maximum(1.0 - cos * cos, 1e-12))
            proj = sinphi * dnorm
            proj = jnp.where(cos < 0.866, 1e8, proj)
            p = jnp.where(colmask, proj, inf)
            dv, iv = [], []
            for k in range(KSEL):
                ml = jnp.min(p, axis=1)
                mv = jnp.min(ml, axis=1, keepdims=True)
                eq = p == mv.reshape(BF, 1, 1)
                fl = jnp.min(jnp.where(eq, flat, big), axis=1)
                mf = jnp.min(fl, axis=1, keepdims=True)
                dv.append(mv)
                iv.append(mf)
                p = jnp.where(flat == mf.reshape(BF, 1, 1), inf, p)
            dout_ref[:, r, :] = jnp.concatenate(dv, axis=1)
            iout_ref[:, r, :] = jnp.concatenate(iv, axis=1)


def _repair(ids, flags, ray_o, ray_d24, pts_t, dist, idx):
    grid_spec = pltpu.PrefetchScalarGridSpec(
        num_scalar_prefetch=2,
        grid=(NF // BF,),
        in_specs=[
            pl.BlockSpec((BF, 3), lambda j, ids, fl: (ids[j], 0)),
            pl.BlockSpec((BF, 24), lambda j, ids, fl: (ids[j], 0)),
            pl.BlockSpec((3, NCH, 128), lambda j, ids, fl: (0, 0, 0)),
            pl.BlockSpec((BF, NRAY, KSEL), lambda j, ids, fl: (ids[j], 0, 0)),
            pl.BlockSpec((BF, NRAY, KSEL), lambda j, ids, fl: (ids[j], 0, 0)),
        ],
        out_specs=[
            pl.BlockSpec((BF, NRAY, KSEL), lambda j, ids, fl: (ids[j], 0, 0)),
            pl.BlockSpec((BF, NRAY, KSEL), lambda j, ids, fl: (ids[j], 0, 0)),
        ],
    )
    return pl.pallas_call(
        _repair_body,
        grid_spec=grid_spec,
        out_shape=[
            jax.ShapeDtypeStruct((NF, NRAY, KSEL), jnp.float32),
            jax.ShapeDtypeStruct((NF, NRAY, KSEL), jnp.int32),
        ],
        input_output_aliases={5: 0, 6: 1},
    )(ids, flags, ray_o, ray_d24, pts_t, dist, idx)


_NC = 2
_NS = 16
_NW = _NC * _NS
_BPW = NSEL // _NW


def _gather_body(pts_hbm, idx_hbm, out_hbm, idx_v, rows_v, sem):
    wid = lax.axis_index("s") * _NC + lax.axis_index("c")
    base = wid * _BPW
    pltpu.sync_copy(idx_hbm.at[pl.ds(base, _BPW)], idx_v)
    pltpu.async_copy(pts_hbm.at[idx_v], rows_v, sem).wait()
    pltpu.sync_copy(rows_v, out_hbm.at[pl.ds(base, _BPW)])


def _gather_sc(pts16, idx_flat):
    return pl.kernel(
        _gather_body,
        out_type=jax.ShapeDtypeStruct((NSEL, DPAD), jnp.float32),
        mesh=plsc.VectorSubcoreMesh(core_axis_name="c", subcore_axis_name="s"),
        scratch_types=[
            pltpu.VMEM((_BPW,), jnp.int32),
            pltpu.VMEM((_BPW, DPAD), jnp.float32),
            pltpu.SemaphoreType.DMA,
        ],
        compiler_params=pltpu.CompilerParams(use_tc_tiling_on_sc=False),
    )(pts16, idx_flat)


def _acos(x):
    a = jnp.abs(x)
    p = jnp.float32(-0.0012624911)
    for coef in (0.0066700901, -0.0170881256, 0.0308918810, -0.0501743046,
                 0.0889789874, -0.2145988016, 1.5707963050):
        p = p * a + jnp.float32(coef)
    r = jnp.sqrt(jnp.maximum(1.0 - a, 0.0)) * p
    return jnp.where(x < 0.0, jnp.float32(math.pi) - r, r)


def _feat_body(gx_ref, gy_ref, gz_ref, ro_ref, dist_ref,
               npd_ref, elev_ref, azim_ref, sky_ref):
    eps = 1e-5
    gx = gx_ref[...] - ro_ref[:, 0:1]
    gy = gy_ref[...] - ro_ref[:, 1:2]
    gz = gz_ref[...] - ro_ref[:, 2:3]
    npd = jnp.sqrt((gx * gx + gy * gy) + gz * gz)
    u1 = gz / (npd + eps)
    elev = _acos(u1)
    sin_elev = jnp.sqrt(jnp.maximum(1.0 - u1 * u1, 0.0))
    az = _acos(gx / (npd * sin_elev + eps))
    az = jnp.where(gy < 0.0, 2.0 * math.pi - az, az)
    npd_ref[...] = npd
    elev_ref[...] = elev
    azim_ref[...] = az
    sky_ref[...] = (dist_ref[...] >= 1e8 - 1).astype(jnp.int32)


def _features(gx, gy, gz, ray_o, dist):
    n = NRAY * KSEL
    return pl.pallas_call(
        _feat_body,
        grid=(1,),
        in_specs=[
            pl.BlockSpec((NF, n), lambda i: (0, 0)),
            pl.BlockSpec((NF, n), lambda i: (0, 0)),
            pl.BlockSpec((NF, n), lambda i: (0, 0)),
            pl.BlockSpec((NF, 3), lambda i: (0, 0)),
            pl.BlockSpec((NF, n), lambda i: (0, 0)),
        ],
        out_specs=[pl.BlockSpec((NF, n), lambda i: (0, 0))] * 4,
        out_shape=[
            jax.ShapeDtypeStruct((NF, n), jnp.float32),
            jax.ShapeDtypeStruct((NF, n), jnp.float32),
            jax.ShapeDtypeStruct((NF, n), jnp.float32),
            jax.ShapeDtypeStruct((NF, n), jnp.int32),
        ],
    )(gx, gy, gz, ray_o, dist)


def kernel(ray_o, ray_d, pts):
    ray_d24 = ray_d.reshape(NF, NRAY * 3)
    pts_t = jnp.pad(pts.T, ((0, 0), (0, NPAD - NPTS))).reshape(3, NCH, 128)
    dist, idx, bad = _dist_topk(ray_o, ray_d24, pts_t)
    gbad = jnp.max(bad.reshape(NF // BF, BF * NRAY), axis=1)
    flags, ids = jax.lax.top_k(gbad, NF // BF)
    dist, idx = _repair(ids, flags, ray_o, ray_d24, pts_t, dist, idx)

    pts16 = jnp.pad(pts, ((0, 0), (0, DPAD - 3)))
    rows = _gather_sc(pts16, idx.reshape(NSEL))

    g = rows[:, :3].reshape(NF, NRAY * KSEL, 3)
    npd, elev, azim, sky = _features(
        g[:, :, 0], g[:, :, 1], g[:, :, 2], ray_o,
        dist.reshape(NF, NRAY * KSEL))

    shp = (NF, NRAY, KSEL, 1)
    return (dist, idx, sky.astype(bool).reshape(NF, NRAY, KSEL),
            npd.reshape(shp), elev.reshape(shp), azim.reshape(shp))


# speedup vs baseline: 5.5044x; 1.2870x over previous
import math

import jax
import jax.numpy as jnp
from jax import lax
from jax.experimental import pallas as pl
from jax.experimental.pallas import tpu as pltpu
from jax.experimental.pallas import tpu_sc as plsc

KSEL = 8
NRAY = 8
NF = 512
NPTS = 10000
NPAD = 10240
BF = 8
NSEL = NF * NRAY * KSEL
DPAD = 16


def _dist_topk_body(ro_ref, rd_ref, pts_ref, dist_ref, idx_ref):
    px = pts_ref[0:1, :]
    py = pts_ref[1:2, :]
    pz = pts_ref[2:3, :]
    ox = ro_ref[:, 0:1]
    oy = ro_ref[:, 1:2]
    oz = ro_ref[:, 2:3]
    dx = px - ox
    dy = py - oy
    dz = pz - oz
    dnorm = jnp.sqrt((dx * dx + dy * dy) + dz * dz)
    maxd = jnp.maximum(dnorm, 1e-12)
    ux = dx / maxd
    uy = dy / maxd
    uz = dz / maxd
    lane = lax.broadcasted_iota(jnp.int32, (BF, NPAD), 1)
    colmask = lane < NPTS
    big = jnp.int32(2 ** 30)
    for r in range(NRAY):
        a = rd_ref[:, 3 * r:3 * r + 1]
        b = rd_ref[:, 3 * r + 1:3 * r + 2]
        c = rd_ref[:, 3 * r + 2:3 * r + 3]
        nrm = jnp.sqrt((a * a + b * b) + c * c)
        nrm = jnp.maximum(nrm, 1e-12)
        an = a / nrm
        bn = b / nrm
        cn = c / nrm
        cos = (an * ux + bn * uy) + cn * uz
        sinphi = jnp.sqrt(jnp.maximum(1.0 - cos * cos, 1e-12))
        proj = sinphi * dnorm
        proj = jnp.where(cos < 0.866, 1e8, proj)
        p = jnp.where(colmask, proj, jnp.inf)
        dvals = []
        ivals = []
        for k in range(KSEL):
            m = jnp.min(p, axis=1, keepdims=True)
            eqm = p == m
            im = jnp.min(jnp.where(eqm, lane, big), axis=1, keepdims=True)
            dvals.append(m)
            ivals.append(im)
            if k < KSEL - 1:
                p = jnp.where(lane == im, jnp.inf, p)
        dist_ref[:, r, :] = jnp.concatenate(dvals, axis=1)
        idx_ref[:, r, :] = jnp.concatenate(ivals, axis=1)


def _dist_topk(ray_o, ray_d24, pts_t):
    return pl.pallas_call(
        _dist_topk_body,
        grid=(NF // BF,),
        in_specs=[
            pl.BlockSpec((BF, 3), lambda i: (i, 0)),
            pl.BlockSpec((BF, 24), lambda i: (i, 0)),
            pl.BlockSpec((3, NPAD), lambda i: (0, 0)),
        ],
        out_specs=[
            pl.BlockSpec((BF, NRAY, KSEL), lambda i: (i, 0, 0)),
            pl.BlockSpec((BF, NRAY, KSEL), lambda i: (i, 0, 0)),
        ],
        out_shape=[
            jax.ShapeDtypeStruct((NF, NRAY, KSEL), jnp.float32),
            jax.ShapeDtypeStruct((NF, NRAY, KSEL), jnp.int32),
        ],
    )(ray_o, ray_d24, pts_t)


_NC = 2
_NS = 16
_NW = _NC * _NS
_BPW = NSEL // _NW


def _gather_body(pts_hbm, idx_hbm, out_hbm, idx_v, rows_v, sem):
    wid = lax.axis_index("s") * _NC + lax.axis_index("c")
    base = wid * _BPW
    pltpu.sync_copy(idx_hbm.at[pl.ds(base, _BPW)], idx_v)
    pltpu.async_copy(pts_hbm.at[idx_v], rows_v, sem).wait()
    pltpu.sync_copy(rows_v, out_hbm.at[pl.ds(base, _BPW)])


def _gather_sc(pts16, idx_flat):
    return pl.kernel(
        _gather_body,
        out_type=jax.ShapeDtypeStruct((NSEL, DPAD), jnp.float32),
        mesh=plsc.VectorSubcoreMesh(core_axis_name="c", subcore_axis_name="s"),
        scratch_types=[
            pltpu.VMEM((_BPW,), jnp.int32),
            pltpu.VMEM((_BPW, DPAD), jnp.float32),
            pltpu.SemaphoreType.DMA,
        ],
        compiler_params=pltpu.CompilerParams(use_tc_tiling_on_sc=False),
    )(pts16, idx_flat)


def _acos(x):
    a = jnp.abs(x)
    p = jnp.float32(-0.0012624911)
    for coef in (0.0066700901, -0.0170881256, 0.0308918810, -0.0501743046,
                 0.0889789874, -0.2145988016, 1.5707963050):
        p = p * a + jnp.float32(coef)
    r = jnp.sqrt(jnp.maximum(1.0 - a, 0.0)) * p
    return jnp.where(x < 0.0, jnp.float32(math.pi) - r, r)


def _feat_body(gx_ref, gy_ref, gz_ref, ro_ref, dist_ref,
               npd_ref, elev_ref, azim_ref, sky_ref):
    eps = 1e-5
    gx = gx_ref[...] - ro_ref[:, 0:1]
    gy = gy_ref[...] - ro_ref[:, 1:2]
    gz = gz_ref[...] - ro_ref[:, 2:3]
    npd = jnp.sqrt((gx * gx + gy * gy) + gz * gz)
    u1 = gz / (npd + eps)
    elev = _acos(u1)
    sin_elev = jnp.sqrt(jnp.maximum(1.0 - u1 * u1, 0.0))
    az = _acos(gx / (npd * sin_elev + eps))
    az = jnp.where(gy < 0.0, 2.0 * math.pi - az, az)
    npd_ref[...] = npd
    elev_ref[...] = elev
    azim_ref[...] = az
    sky_ref[...] = (dist_ref[...] >= 1e8 - 1).astype(jnp.int32)


def _features(gx, gy, gz, ray_o, dist):
    n = NRAY * KSEL
    return pl.pallas_call(
        _feat_body,
        grid=(1,),
        in_specs=[
            pl.BlockSpec((NF, n), lambda i: (0, 0)),
            pl.BlockSpec((NF, n), lambda i: (0, 0)),
            pl.BlockSpec((NF, n), lambda i: (0, 0)),
            pl.BlockSpec((NF, 3), lambda i: (0, 0)),
            pl.BlockSpec((NF, n), lambda i: (0, 0)),
        ],
        out_specs=[pl.BlockSpec((NF, n), lambda i: (0, 0))] * 4,
        out_shape=[
            jax.ShapeDtypeStruct((NF, n), jnp.float32),
            jax.ShapeDtypeStruct((NF, n), jnp.float32),
            jax.ShapeDtypeStruct((NF, n), jnp.float32),
            jax.ShapeDtypeStruct((NF, n), jnp.int32),
        ],
    )(gx, gy, gz, ray_o, dist)


def kernel(ray_o, ray_d, pts):
    ray_d24 = ray_d.reshape(NF, NRAY * 3)
    pts_t = jnp.pad(pts.T, ((0, 0), (0, NPAD - NPTS)))
    dist, idx = _dist_topk(ray_o, ray_d24, pts_t)

    pts16 = jnp.pad(pts, ((0, 0), (0, DPAD - 3)))
    rows = _gather_sc(pts16, idx.reshape(NSEL))

    g = rows[:, :3].reshape(NF, NRAY * KSEL, 3)
    npd, elev, azim, sky = _features(
        g[:, :, 0], g[:, :, 1], g[:, :, 2], ray_o,
        dist.reshape(NF, NRAY * KSEL))

    shp = (NF, NRAY, KSEL, 1)
    return (dist, idx, sky.astype(bool).reshape(NF, NRAY, KSEL),
            npd.reshape(shp), elev.reshape(shp), azim.reshape(shp))
